# Initial kernel scaffold; baseline (speedup 1.0000x reference)
import jax
import jax.numpy as jnp
from jax.experimental import pallas as pl


def kernel(x, edge_index, W1, b1, W2, b2):
    raise NotImplementedError("write your pallas kernel here")


# speedup vs baseline: 4.5526x; 4.5526x over previous
import jax
import jax.numpy as jnp
from jax import lax
from jax.experimental import pallas as pl
from jax.experimental.pallas import tpu as pltpu
from jax.experimental.pallas import tpu_sc as plsc

N = 10000
D = 128
E = 320000
NPAD = 10240
EPAD = 327680
NRH = 5000
SH_ROWS = 5120
SH_TR = SH_ROWS // 16
JUNK = SH_ROWS - 1
EPT = EPAD // 16
CHUNK = 128
NCHUNK = EPT // CHUNK
BR = 1024

_mesh = plsc.VectorSubcoreMesh(core_axis_name="c", subcore_axis_name="s")


def _remap_dst(dstb, nbase):
    for k in range(CHUNK // 16):
        d16 = dstb[pl.ds(k * 16, 16)]
        local = d16 - nbase
        ok = (local >= 0) & (local < NRH)
        dstb[pl.ds(k * 16, 16)] = jnp.where(ok, local, JUNK)


def _deg_body(dst_hbm, ones_hbm, zeros_hbm, deg_out, dstb, ones_v, buf_v,
              deg_sh):
    cid = lax.axis_index("c")
    sid = lax.axis_index("s")
    nbase = cid * NRH
    ebase = sid * EPT
    rbase = sid * SH_TR
    pltpu.sync_copy(ones_hbm, ones_v)
    pltpu.sync_copy(zeros_hbm, buf_v)
    pltpu.sync_copy(buf_v, deg_sh.at[pl.ds(rbase, SH_TR)])
    plsc.subcore_barrier()

    @pl.loop(0, NCHUNK)
    def _(c):
        pltpu.sync_copy(dst_hbm.at[pl.ds(ebase + c * CHUNK, CHUNK)], dstb)
        _remap_dst(dstb, nbase)
        pltpu.sync_copy(ones_v, deg_sh.at[dstb], add=True)

    plsc.subcore_barrier()
    pltpu.sync_copy(deg_sh.at[pl.ds(rbase, SH_TR)], buf_v)
    pltpu.sync_copy(buf_v, deg_out.at[pl.ds(cid * SH_ROWS + rbase, SH_TR)])


@jax.jit
def _deg_call(dst, ones, zeros):
    return pl.kernel(
        _deg_body,
        out_type=jax.ShapeDtypeStruct((2 * SH_ROWS, D), jnp.float32),
        mesh=_mesh,
        scratch_types=[
            pltpu.VMEM((CHUNK,), jnp.int32),
            pltpu.VMEM((CHUNK, D), jnp.float32),
            pltpu.VMEM((SH_TR, D), jnp.float32),
            pltpu.VMEM_SHARED((SH_ROWS, D), jnp.float32),
        ],
    )(dst, ones, zeros)


def _edge_body(src_hbm, dst_hbm, hs_hbm, acc_out, srcb, dstb, rows_v, buf_v,
               acc_sh, gsem):
    cid = lax.axis_index("c")
    sid = lax.axis_index("s")
    nbase = cid * NRH
    ebase = sid * EPT
    rbase = sid * SH_TR
    pltpu.sync_copy(hs_hbm.at[pl.ds(nbase + rbase, SH_TR)], buf_v)
    pltpu.sync_copy(buf_v, acc_sh.at[pl.ds(rbase, SH_TR)])
    plsc.subcore_barrier()

    @pl.loop(0, NCHUNK)
    def _(c):
        off = ebase + c * CHUNK
        pltpu.sync_copy(src_hbm.at[pl.ds(off, CHUNK)], srcb)
        pltpu.sync_copy(dst_hbm.at[pl.ds(off, CHUNK)], dstb)
        _remap_dst(dstb, nbase)
        pltpu.async_copy(hs_hbm.at[srcb], rows_v, gsem).wait()
        pltpu.sync_copy(rows_v, acc_sh.at[dstb], add=True)

    plsc.subcore_barrier()
    pltpu.sync_copy(acc_sh.at[pl.ds(rbase, SH_TR)], buf_v)
    pltpu.sync_copy(buf_v, acc_out.at[pl.ds(cid * SH_ROWS + rbase, SH_TR)])


@jax.jit
def _edge_call(src, dst, hs):
    return pl.kernel(
        _edge_body,
        out_type=jax.ShapeDtypeStruct((2 * SH_ROWS, D), jnp.float32),
        mesh=_mesh,
        scratch_types=[
            pltpu.VMEM((CHUNK,), jnp.int32),
            pltpu.VMEM((CHUNK,), jnp.int32),
            pltpu.VMEM((CHUNK, D), jnp.float32),
            pltpu.VMEM((SH_TR, D), jnp.float32),
            pltpu.VMEM_SHARED((SH_ROWS, D), jnp.float32),
            pltpu.SemaphoreType.DMA,
        ],
    )(src, dst, hs)


def _mm1_body(x_ref, w_ref, deg_ref, hs_ref):
    dinv = lax.rsqrt(deg_ref[...] + 1.0)
    hs_ref[...] = jnp.dot(x_ref[...], w_ref[...],
                          preferred_element_type=jnp.float32,
                          precision=lax.Precision.HIGHEST) * dinv


def _mid_body(acc_ref, deg_ref, b1_ref, w2_ref, hs2_ref):
    dinv = lax.rsqrt(deg_ref[...] + 1.0)
    h1 = jnp.maximum(acc_ref[...] * dinv + b1_ref[...], 0.0)
    hs2_ref[...] = jnp.dot(h1, w2_ref[...],
                           preferred_element_type=jnp.float32,
                           precision=lax.Precision.HIGHEST) * dinv


def _fin_body(acc_ref, deg_ref, b2_ref, out_ref):
    dinv = lax.rsqrt(deg_ref[...] + 1.0)
    out_ref[...] = acc_ref[...] * dinv + b2_ref[...]


_row_spec = pl.BlockSpec((BR, D), lambda i: (i, 0))
_col_spec = pl.BlockSpec((BR, 1), lambda i: (i, 0))
_w_spec = pl.BlockSpec((D, D), lambda i: (0, 0))
_b_spec = pl.BlockSpec((1, D), lambda i: (0, 0))
_GRID = (NPAD // BR,)
_row_out = jax.ShapeDtypeStruct((NPAD, D), jnp.float32)

_mm1_call = pl.pallas_call(
    _mm1_body, grid=_GRID,
    in_specs=[_row_spec, _w_spec, _col_spec],
    out_specs=_row_spec, out_shape=_row_out)

_mid_call = pl.pallas_call(
    _mid_body, grid=_GRID,
    in_specs=[_row_spec, _col_spec, _b_spec, _w_spec],
    out_specs=_row_spec, out_shape=_row_out)

_fin_call = pl.pallas_call(
    _fin_body, grid=_GRID,
    in_specs=[_row_spec, _col_spec, _b_spec],
    out_specs=_row_spec, out_shape=_row_out)


def _cat_halves(t):
    return jnp.concatenate(
        [t[0:NRH], t[SH_ROWS:SH_ROWS + NRH],
         jnp.zeros((NPAD - N, D), jnp.float32)])


def kernel(x, edge_index, W1, b1, W2, b2):
    src = edge_index[0].astype(jnp.int32)
    dst = edge_index[1].astype(jnp.int32)
    src_p = jnp.concatenate([src, jnp.zeros((EPAD - E,), jnp.int32)])
    dst_p = jnp.concatenate([dst, jnp.full((EPAD - E,), N, jnp.int32)])
    x_p = jnp.concatenate([x, jnp.zeros((NPAD - N, D), x.dtype)])
    ones = jnp.ones((CHUNK, D), jnp.float32)
    zeros = jnp.zeros((SH_TR, D), jnp.float32)

    deg = _cat_halves(_deg_call(dst_p, ones, zeros))[:, 0:1]
    hs1 = _mm1_call(x_p, W1, deg)
    acc1 = _cat_halves(_edge_call(src_p, dst_p, hs1))
    hs2 = _mid_call(acc1, deg, b1.reshape(1, D), W2)
    acc2 = _cat_halves(_edge_call(src_p, dst_p, hs2))
    out = _fin_call(acc2, deg, b2.reshape(1, D))
    return out[:N]


# speedup vs baseline: 5.2658x; 1.1567x over previous
import jax
import jax.numpy as jnp
from jax import lax
from jax.experimental import pallas as pl
from jax.experimental.pallas import tpu as pltpu
from jax.experimental.pallas import tpu_sc as plsc

N = 10000
D = 128
E = 320000
NPAD = 10240
EPAD = 327680
NRH = 5000
SH_ROWS = 5120
SH_TR = SH_ROWS // 16
JUNK = SH_ROWS - 1
EPT = EPAD // 16
CHUNK = 128
NCHUNK = EPT // CHUNK
BR = 1024

_mesh = plsc.VectorSubcoreMesh(core_axis_name="c", subcore_axis_name="s")


def _remap_dst(dstb, nbase):
    for k in range(CHUNK // 16):
        d16 = dstb[pl.ds(k * 16, 16)]
        local = d16 - nbase
        ok = (local >= 0) & (local < NRH)
        dstb[pl.ds(k * 16, 16)] = jnp.where(ok, local, JUNK)


def _deg_body(dst_hbm, ones_hbm, zeros_hbm, deg_out, dstb, ones_v, buf_v,
              deg_sh):
    cid = lax.axis_index("c")
    sid = lax.axis_index("s")
    nbase = cid * NRH
    ebase = sid * EPT
    rbase = sid * SH_TR
    pltpu.sync_copy(ones_hbm, ones_v)
    pltpu.sync_copy(zeros_hbm, buf_v)
    pltpu.sync_copy(buf_v, deg_sh.at[pl.ds(rbase, SH_TR)])
    plsc.subcore_barrier()

    @pl.loop(0, NCHUNK)
    def _(c):
        pltpu.sync_copy(dst_hbm.at[pl.ds(ebase + c * CHUNK, CHUNK)], dstb)
        _remap_dst(dstb, nbase)
        pltpu.sync_copy(ones_v, deg_sh.at[dstb], add=True)

    plsc.subcore_barrier()
    pltpu.sync_copy(deg_sh.at[pl.ds(rbase, SH_TR)], buf_v)
    pltpu.sync_copy(buf_v, deg_out.at[pl.ds(cid * SH_ROWS + rbase, SH_TR)])


@jax.jit
def _deg_call(dst, ones, zeros):
    return pl.kernel(
        _deg_body,
        out_type=jax.ShapeDtypeStruct((2 * SH_ROWS, D), jnp.float32),
        mesh=_mesh,
        scratch_types=[
            pltpu.VMEM((CHUNK,), jnp.int32),
            pltpu.VMEM((CHUNK, D), jnp.float32),
            pltpu.VMEM((SH_TR, D), jnp.float32),
            pltpu.VMEM_SHARED((SH_ROWS, D), jnp.float32),
        ],
    )(dst, ones, zeros)


NBUF = 2
IB = 64


def _edge_body(src_hbm, dst_hbm, hs_hbm, acc_out, src_all, dst_all, rows_v,
               buf_v, acc_sh, gsem):
    cid = lax.axis_index("c")
    sid = lax.axis_index("s")
    nbase = cid * NRH
    ebase = sid * EPT
    rbase = sid * SH_TR
    pltpu.sync_copy(src_hbm.at[pl.ds(ebase, EPT)], src_all)
    pltpu.sync_copy(dst_hbm.at[pl.ds(sid * NCHUNK, NCHUNK)], dst_all)
    for b in range(SH_TR // IB):
        pltpu.sync_copy(hs_hbm.at[pl.ds(nbase + rbase + b * IB, IB)], buf_v)
        pltpu.sync_copy(buf_v, acc_sh.at[pl.ds(rbase + b * IB, IB)])
    plsc.subcore_barrier()

    def _gather(c, buf):
        pltpu.async_copy(hs_hbm.at[src_all.at[pl.ds(c * CHUNK, CHUNK)]],
                         rows_v.at[buf], gsem)

    for p in range(NBUF):
        _gather(p, p)

    @pl.loop(0, NCHUNK, step=NBUF)
    def _(c):
        for b in range(NBUF):
            for k in range(CHUNK // 16):
                d16 = dst_all[c + b, pl.ds(k * 16, 16)]
                local = d16 - nbase
                ok = (local >= 0) & (local < NRH)
                dst_all[c + b, pl.ds(k * 16, 16)] = jnp.where(ok, local, JUNK)
            pltpu.make_async_copy(hs_hbm.at[pl.ds(0, CHUNK)], rows_v.at[b],
                                  gsem).wait()
            pltpu.sync_copy(rows_v.at[b], acc_sh.at[dst_all.at[c + b]],
                            add=True)

            @pl.when(c + b + NBUF < NCHUNK)
            def _():
                _gather(c + b + NBUF, b)

    plsc.subcore_barrier()
    for b in range(SH_TR // IB):
        pltpu.sync_copy(acc_sh.at[pl.ds(rbase + b * IB, IB)], buf_v)
        pltpu.sync_copy(buf_v,
                        acc_out.at[pl.ds(cid * SH_ROWS + rbase + b * IB, IB)])


@jax.jit
def _edge_call(src, dst, hs):
    return pl.kernel(
        _edge_body,
        out_type=jax.ShapeDtypeStruct((2 * SH_ROWS, D), jnp.float32),
        mesh=_mesh,
        scratch_types=[
            pltpu.VMEM((EPT,), jnp.int32),
            pltpu.VMEM((NCHUNK, CHUNK), jnp.int32),
            pltpu.VMEM((NBUF, CHUNK, D), jnp.float32),
            pltpu.VMEM((IB, D), jnp.float32),
            pltpu.VMEM_SHARED((SH_ROWS, D), jnp.float32),
            pltpu.SemaphoreType.DMA,
        ],
    )(src, dst, hs)


def _mm1_body(x_ref, w_ref, deg_ref, hs_ref):
    dinv = lax.rsqrt(deg_ref[...] + 1.0)
    hs_ref[...] = jnp.dot(x_ref[...], w_ref[...],
                          preferred_element_type=jnp.float32,
                          precision=lax.Precision.HIGHEST) * dinv


def _mid_body(acc_ref, deg_ref, b1_ref, w2_ref, hs2_ref):
    dinv = lax.rsqrt(deg_ref[...] + 1.0)
    h1 = jnp.maximum(acc_ref[...] * dinv + b1_ref[...], 0.0)
    hs2_ref[...] = jnp.dot(h1, w2_ref[...],
                           preferred_element_type=jnp.float32,
                           precision=lax.Precision.HIGHEST) * dinv


def _fin_body(acc_ref, deg_ref, b2_ref, out_ref):
    dinv = lax.rsqrt(deg_ref[...] + 1.0)
    out_ref[...] = acc_ref[...] * dinv + b2_ref[...]


_row_spec = pl.BlockSpec((BR, D), lambda i: (i, 0))
_col_spec = pl.BlockSpec((BR, 1), lambda i: (i, 0))
_w_spec = pl.BlockSpec((D, D), lambda i: (0, 0))
_b_spec = pl.BlockSpec((1, D), lambda i: (0, 0))
_GRID = (NPAD // BR,)
_row_out = jax.ShapeDtypeStruct((NPAD, D), jnp.float32)

_mm1_call = pl.pallas_call(
    _mm1_body, grid=_GRID,
    in_specs=[_row_spec, _w_spec, _col_spec],
    out_specs=_row_spec, out_shape=_row_out)

_mid_call = pl.pallas_call(
    _mid_body, grid=_GRID,
    in_specs=[_row_spec, _col_spec, _b_spec, _w_spec],
    out_specs=_row_spec, out_shape=_row_out)

_fin_call = pl.pallas_call(
    _fin_body, grid=_GRID,
    in_specs=[_row_spec, _col_spec, _b_spec],
    out_specs=_row_spec, out_shape=_row_out)


def _cat_halves(t):
    return jnp.concatenate(
        [t[0:NRH], t[SH_ROWS:SH_ROWS + NRH],
         jnp.zeros((NPAD - N, D), jnp.float32)])


def kernel(x, edge_index, W1, b1, W2, b2):
    src = edge_index[0].astype(jnp.int32)
    dst = edge_index[1].astype(jnp.int32)
    src_p = jnp.concatenate([src, jnp.zeros((EPAD - E,), jnp.int32)])
    dst_p = jnp.concatenate([dst, jnp.full((EPAD - E,), N, jnp.int32)])
    x_p = jnp.concatenate([x, jnp.zeros((NPAD - N, D), x.dtype)])
    ones = jnp.ones((CHUNK, D), jnp.float32)
    zeros = jnp.zeros((SH_TR, D), jnp.float32)

    dst2d = dst_p.reshape(EPAD // CHUNK, CHUNK)

    deg = _cat_halves(_deg_call(dst_p, ones, zeros))[:, 0:1]
    hs1 = _mm1_call(x_p, W1, deg)
    acc1 = _cat_halves(_edge_call(src_p, dst2d, hs1))
    hs2 = _mid_call(acc1, deg, b1.reshape(1, D), W2)
    acc2 = _cat_halves(_edge_call(src_p, dst2d, hs2))
    out = _fin_call(acc2, deg, b2.reshape(1, D))
    return out[:N]


# speedup vs baseline: 13.8377x; 2.6278x over previous
import dataclasses

import jax
import jax.numpy as jnp
from jax import lax
from jax.experimental import pallas as pl
from jax.experimental.pallas import tpu as pltpu
from jax.experimental.pallas import tpu_sc as plsc

N = 10000
D = 128
E = 320000
NPAD = 10240
EPAD = 327680
NRH = 5000
SH_ROWS = 5120
SH_TR = SH_ROWS // 16
JUNK = SH_ROWS - 1
EPT = EPAD // 16
CHUNK = 128
NCHUNK = EPT // CHUNK
BR = 1024

_mesh = plsc.VectorSubcoreMesh(core_axis_name="c", subcore_axis_name="s")

_sc_params = pltpu.CompilerParams()
if "needs_layout_passes" in pltpu.CompilerParams.__dataclass_fields__:
    _sc_params = dataclasses.replace(_sc_params, needs_layout_passes=False)


def _remap_dst(dstb, nbase):
    for k in range(CHUNK // 16):
        d16 = dstb[pl.ds(k * 16, 16)]
        local = d16 - nbase
        ok = (local >= 0) & (local < NRH)
        dstb[pl.ds(k * 16, 16)] = jnp.where(ok, local, JUNK)


def _deg_body(dst_hbm, ones_hbm, zeros_hbm, deg_out, dstb, ones_v, buf_v,
              deg_sh):
    cid = lax.axis_index("c")
    sid = lax.axis_index("s")
    nbase = cid * NRH
    ebase = sid * EPT
    rbase = sid * SH_TR
    pltpu.sync_copy(ones_hbm, ones_v)
    pltpu.sync_copy(zeros_hbm, buf_v)
    pltpu.sync_copy(buf_v, deg_sh.at[pl.ds(rbase, SH_TR)])
    plsc.subcore_barrier()

    @pl.loop(0, NCHUNK)
    def _(c):
        pltpu.sync_copy(dst_hbm.at[pl.ds(ebase + c * CHUNK, CHUNK)], dstb)
        _remap_dst(dstb, nbase)
        pltpu.sync_copy(ones_v, deg_sh.at[dstb], add=True)

    plsc.subcore_barrier()
    pltpu.sync_copy(deg_sh.at[pl.ds(rbase, SH_TR)], buf_v)
    pltpu.sync_copy(buf_v, deg_out.at[pl.ds(cid * SH_ROWS + rbase, SH_TR)])


@jax.jit
def _deg_call(dst, ones, zeros):
    return pl.kernel(
        _deg_body,
        out_type=jax.ShapeDtypeStruct((2 * SH_ROWS, D), jnp.float32),
        mesh=_mesh,
        scratch_types=[
            pltpu.VMEM((CHUNK,), jnp.int32),
            pltpu.VMEM((CHUNK, D), jnp.float32),
            pltpu.VMEM((SH_TR, D), jnp.float32),
            pltpu.VMEM_SHARED((SH_ROWS, D), jnp.float32),
        ],
    )(dst, ones, zeros)


NBUF = 2
IB = 64
TRASH = EPT + CHUNK


def _edge_body(src_hbm, dst_hbm, hs_hbm, acc_out, src_all, dst_all, dstb,
               rows_v, buf_v, acc_sh, gsem, offr):
    cid = lax.axis_index("c")
    sid = lax.axis_index("s")
    nbase = cid * NRH
    ebase = sid * EPT
    rbase = sid * SH_TR
    pltpu.sync_copy(src_hbm.at[pl.ds(ebase, EPT)], src_all.at[pl.ds(0, EPT)])
    pltpu.sync_copy(dst_hbm.at[pl.ds(ebase, EPT)], dst_all.at[pl.ds(0, EPT)])
    for b in range(SH_TR // IB):
        pltpu.sync_copy(hs_hbm.at[pl.ds(nbase + rbase + b * IB, IB)], buf_v)
        pltpu.sync_copy(buf_v, acc_sh.at[pl.ds(rbase + b * IB, IB)])

    offr[0] = 0
    lane_g = lax.iota(jnp.int32, 16)

    @pl.loop(0, EPT // 16)
    def _(g):
        off = offr[0]
        s16 = src_all[pl.ds(g * 16, 16)]
        d16 = dst_all[pl.ds(g * 16, 16)]
        local = d16 - nbase
        ok = (local >= 0) & (local < NRH)
        oki = ok.astype(jnp.int32)
        pos = off + plsc.cumsum(oki) - oki
        tgt = jnp.where(ok, pos, TRASH)
        plsc.store_scatter(src_all, [tgt], s16)
        plsc.store_scatter(dst_all, [tgt], local)
        offr[0] = off + jnp.sum(oki)

    cnt = offr[0]
    lane = lax.iota(jnp.int32, 16)
    for k in range(CHUNK // 16):
        plsc.store_scatter(src_all, [cnt + k * 16 + lane],
                           jnp.zeros((16,), jnp.int32))
        plsc.store_scatter(dst_all, [cnt + k * 16 + lane],
                           jnp.full((16,), JUNK, jnp.int32))
    nch = (cnt + CHUNK - 1) // CHUNK
    plsc.subcore_barrier()

    def _gather(c, buf):
        pltpu.async_copy(hs_hbm.at[src_all.at[pl.ds(c * CHUNK, CHUNK)]],
                         rows_v.at[buf], gsem)

    for p in range(NBUF):
        @pl.when(p < nch)
        def _():
            _gather(p, p)

    @pl.loop(0, NCHUNK, step=NBUF)
    def _(c):
        for b in range(NBUF):
            @pl.when(c + b < nch)
            def _():
                for k in range(CHUNK // 16):
                    dstb[pl.ds(k * 16, 16)] = (
                        dst_all[pl.ds((c + b) * CHUNK + k * 16, 16)])
                pltpu.make_async_copy(hs_hbm.at[pl.ds(0, CHUNK)],
                                      rows_v.at[b], gsem).wait()
                pltpu.sync_copy(rows_v.at[b], acc_sh.at[dstb], add=True)

                @pl.when(c + b + NBUF < nch)
                def _():
                    _gather(c + b + NBUF, b)

    plsc.subcore_barrier()
    for b in range(SH_TR // IB):
        pltpu.sync_copy(acc_sh.at[pl.ds(rbase + b * IB, IB)], buf_v)
        pltpu.sync_copy(buf_v,
                        acc_out.at[pl.ds(cid * SH_ROWS + rbase + b * IB, IB)])


@jax.jit
def _edge_call(src, dst, hs):
    return pl.kernel(
        _edge_body,
        out_type=jax.ShapeDtypeStruct((2 * SH_ROWS, D), jnp.float32),
        mesh=_mesh,
        scratch_types=[
            pltpu.VMEM((EPT + CHUNK + 16,), jnp.int32),
            pltpu.VMEM((EPT + CHUNK + 16,), jnp.int32),
            pltpu.VMEM((CHUNK,), jnp.int32),
            pltpu.VMEM((NBUF, CHUNK, D), jnp.float32),
            pltpu.VMEM((IB, D), jnp.float32),
            pltpu.VMEM_SHARED((SH_ROWS, D), jnp.float32),
            pltpu.SemaphoreType.DMA,
            pltpu.SMEM((1,), jnp.int32),
        ],
        compiler_params=_sc_params,
    )(src, dst, hs)


def _mm1_body(x_ref, w_ref, deg_ref, hs_ref):
    dinv = lax.rsqrt(deg_ref[...] + 1.0)
    hs_ref[...] = jnp.dot(x_ref[...], w_ref[...],
                          preferred_element_type=jnp.float32,
                          precision=lax.Precision.HIGHEST) * dinv


def _mid_body(acc_ref, deg_ref, b1_ref, w2_ref, hs2_ref):
    dinv = lax.rsqrt(deg_ref[...] + 1.0)
    h1 = jnp.maximum(acc_ref[...] * dinv + b1_ref[...], 0.0)
    hs2_ref[...] = jnp.dot(h1, w2_ref[...],
                           preferred_element_type=jnp.float32,
                           precision=lax.Precision.HIGHEST) * dinv


def _fin_body(acc_ref, deg_ref, b2_ref, out_ref):
    dinv = lax.rsqrt(deg_ref[...] + 1.0)
    out_ref[...] = acc_ref[...] * dinv + b2_ref[...]


_row_spec = pl.BlockSpec((BR, D), lambda i: (i, 0))
_col_spec = pl.BlockSpec((BR, 1), lambda i: (i, 0))
_w_spec = pl.BlockSpec((D, D), lambda i: (0, 0))
_b_spec = pl.BlockSpec((1, D), lambda i: (0, 0))
_GRID = (NPAD // BR,)
_row_out = jax.ShapeDtypeStruct((NPAD, D), jnp.float32)

_mm1_call = pl.pallas_call(
    _mm1_body, grid=_GRID,
    in_specs=[_row_spec, _w_spec, _col_spec],
    out_specs=_row_spec, out_shape=_row_out)

_mid_call = pl.pallas_call(
    _mid_body, grid=_GRID,
    in_specs=[_row_spec, _col_spec, _b_spec, _w_spec],
    out_specs=_row_spec, out_shape=_row_out)

_fin_call = pl.pallas_call(
    _fin_body, grid=_GRID,
    in_specs=[_row_spec, _col_spec, _b_spec],
    out_specs=_row_spec, out_shape=_row_out)


def _cat_halves(t):
    return jnp.concatenate(
        [t[0:NRH], t[SH_ROWS:SH_ROWS + NRH],
         jnp.zeros((NPAD - N, D), jnp.float32)])


def kernel(x, edge_index, W1, b1, W2, b2):
    src = edge_index[0].astype(jnp.int32)
    dst = edge_index[1].astype(jnp.int32)
    src_p = jnp.concatenate([src, jnp.zeros((EPAD - E,), jnp.int32)])
    dst_p = jnp.concatenate([dst, jnp.full((EPAD - E,), N, jnp.int32)])
    x_p = jnp.concatenate([x, jnp.zeros((NPAD - N, D), x.dtype)])
    ones = jnp.ones((CHUNK, D), jnp.float32)
    zeros = jnp.zeros((SH_TR, D), jnp.float32)

    deg = _cat_halves(_deg_call(dst_p, ones, zeros))[:, 0:1]
    hs1 = _mm1_call(x_p, W1, deg)
    acc1 = _cat_halves(_edge_call(src_p, dst_p, hs1))
    hs2 = _mid_call(acc1, deg, b1.reshape(1, D), W2)
    acc2 = _cat_halves(_edge_call(src_p, dst_p, hs2))
    out = _fin_call(acc2, deg, b2.reshape(1, D))
    return out[:N]


# speedup vs baseline: 18.8605x; 1.3630x over previous
import dataclasses

import jax
import jax.numpy as jnp
from jax import lax
from jax.experimental import pallas as pl
from jax.experimental.pallas import tpu as pltpu
from jax.experimental.pallas import tpu_sc as plsc

N = 10000
D = 128
E = 320000
NPAD = 10240
EPAD = 327680
NRH = 5000
SH_ROWS = 5120
SH_TR = SH_ROWS // 16
JUNK = SH_ROWS - 1
EPT = EPAD // 16
CHUNK = 128
NCHUNK = EPT // CHUNK
BR = 1024

_mesh = plsc.VectorSubcoreMesh(core_axis_name="c", subcore_axis_name="s")

_sc_params = pltpu.CompilerParams()
if "needs_layout_passes" in pltpu.CompilerParams.__dataclass_fields__:
    _sc_params = dataclasses.replace(_sc_params, needs_layout_passes=False)


DHR = SH_ROWS // D


def _deg_body(dst_hbm, zeros_hbm, iota_hbm, deg_out, dst_all, hist, idxr,
              deg_sh):
    cid = lax.axis_index("c")
    sid = lax.axis_index("s")
    nbase = cid * NRH
    ebase = sid * EPT
    pltpu.sync_copy(dst_hbm.at[pl.ds(ebase, EPT)], dst_all)
    pltpu.sync_copy(zeros_hbm, hist)
    pltpu.sync_copy(iota_hbm, idxr)

    @pl.when(sid == 0)
    def _():
        pltpu.sync_copy(zeros_hbm, deg_sh)

    plsc.subcore_barrier()
    ones16 = jnp.ones((16,), jnp.float32)

    @pl.loop(0, EPT // 16)
    def _(g):
        d16 = dst_all[pl.ds(g * 16, 16)]
        local = d16 - nbase
        ok = (local >= 0) & (local < NRH)
        lcl = jnp.where(ok, local, JUNK)
        plsc.addupdate_scatter(hist, [lcl >> 7, lcl & 127], ones16)

    pltpu.sync_copy(hist, deg_sh.at[idxr], add=True)
    plsc.subcore_barrier()

    @pl.when(sid == 0)
    def _():
        pltpu.sync_copy(deg_sh, hist)
        pltpu.sync_copy(hist, deg_out.at[pl.ds(cid * DHR, DHR)])


@jax.jit
def _deg_call(dst, zeros, iota40):
    return pl.kernel(
        _deg_body,
        out_type=jax.ShapeDtypeStruct((2 * DHR, D), jnp.float32),
        mesh=_mesh,
        scratch_types=[
            pltpu.VMEM((EPT,), jnp.int32),
            pltpu.VMEM((DHR, D), jnp.float32),
            pltpu.VMEM((DHR,), jnp.int32),
            pltpu.VMEM_SHARED((DHR, D), jnp.float32),
        ],
        compiler_params=_sc_params,
    )(dst, zeros, iota40)


NBUF = 2
IB = 64
TRASH = EPT + CHUNK


def _edge_body(src_hbm, dst_hbm, hs_hbm, acc_out, src_all, dst_all, dstb,
               rows_v, buf_v, acc_sh, gsem, offr):
    cid = lax.axis_index("c")
    sid = lax.axis_index("s")
    nbase = cid * NRH
    ebase = sid * EPT
    rbase = sid * SH_TR
    pltpu.sync_copy(src_hbm.at[pl.ds(ebase, EPT)], src_all.at[pl.ds(0, EPT)])
    pltpu.sync_copy(dst_hbm.at[pl.ds(ebase, EPT)], dst_all.at[pl.ds(0, EPT)])
    for b in range(SH_TR // IB):
        pltpu.sync_copy(hs_hbm.at[pl.ds(nbase + rbase + b * IB, IB)], buf_v)
        pltpu.sync_copy(buf_v, acc_sh.at[pl.ds(rbase + b * IB, IB)])

    offr[0] = 0
    lane_g = lax.iota(jnp.int32, 16)

    @pl.loop(0, EPT // 16)
    def _(g):
        off = offr[0]
        s16 = src_all[pl.ds(g * 16, 16)]
        d16 = dst_all[pl.ds(g * 16, 16)]
        local = d16 - nbase
        ok = (local >= 0) & (local < NRH)
        oki = ok.astype(jnp.int32)
        pos = off + plsc.cumsum(oki) - oki
        tgt = jnp.where(ok, pos, TRASH)
        plsc.store_scatter(src_all, [tgt], s16)
        plsc.store_scatter(dst_all, [tgt], local)
        offr[0] = off + jnp.sum(oki)

    cnt = offr[0]
    lane = lax.iota(jnp.int32, 16)
    for k in range(CHUNK // 16):
        plsc.store_scatter(src_all, [cnt + k * 16 + lane],
                           jnp.zeros((16,), jnp.int32))
        plsc.store_scatter(dst_all, [cnt + k * 16 + lane],
                           jnp.full((16,), JUNK, jnp.int32))
    nch = (cnt + CHUNK - 1) // CHUNK
    plsc.subcore_barrier()

    def _gather(c, buf):
        pltpu.async_copy(hs_hbm.at[src_all.at[pl.ds(c * CHUNK, CHUNK)]],
                         rows_v.at[buf], gsem)

    for p in range(NBUF):
        @pl.when(p < nch)
        def _():
            _gather(p, p)

    @pl.loop(0, NCHUNK, step=NBUF)
    def _(c):
        for b in range(NBUF):
            @pl.when(c + b < nch)
            def _():
                for k in range(CHUNK // 16):
                    dstb[pl.ds(k * 16, 16)] = (
                        dst_all[pl.ds((c + b) * CHUNK + k * 16, 16)])
                pltpu.make_async_copy(hs_hbm.at[pl.ds(0, CHUNK)],
                                      rows_v.at[b], gsem).wait()
                pltpu.sync_copy(rows_v.at[b], acc_sh.at[dstb], add=True)

                @pl.when(c + b + NBUF < nch)
                def _():
                    _gather(c + b + NBUF, b)

    plsc.subcore_barrier()
    for b in range(SH_TR // IB):
        pltpu.sync_copy(acc_sh.at[pl.ds(rbase + b * IB, IB)], buf_v)
        pltpu.sync_copy(buf_v,
                        acc_out.at[pl.ds(cid * SH_ROWS + rbase + b * IB, IB)])


@jax.jit
def _edge_call(src, dst, hs):
    return pl.kernel(
        _edge_body,
        out_type=jax.ShapeDtypeStruct((2 * SH_ROWS, D), jnp.float32),
        mesh=_mesh,
        scratch_types=[
            pltpu.VMEM((EPT + CHUNK + 16,), jnp.int32),
            pltpu.VMEM((EPT + CHUNK + 16,), jnp.int32),
            pltpu.VMEM((CHUNK,), jnp.int32),
            pltpu.VMEM((NBUF, CHUNK, D), jnp.float32),
            pltpu.VMEM((IB, D), jnp.float32),
            pltpu.VMEM_SHARED((SH_ROWS, D), jnp.float32),
            pltpu.SemaphoreType.DMA,
            pltpu.SMEM((1,), jnp.int32),
        ],
        compiler_params=_sc_params,
    )(src, dst, hs)


def _mm1_body(x_ref, w_ref, deg_ref, hs_ref):
    dinv = lax.rsqrt(deg_ref[...] + 1.0)
    hs_ref[...] = jnp.dot(x_ref[...], w_ref[...],
                          preferred_element_type=jnp.float32,
                          precision=lax.Precision.HIGHEST) * dinv


def _mid_body(acc_ref, deg_ref, b1_ref, w2_ref, hs2_ref):
    dinv = lax.rsqrt(deg_ref[...] + 1.0)
    h1 = jnp.maximum(acc_ref[...] * dinv + b1_ref[...], 0.0)
    hs2_ref[...] = jnp.dot(h1, w2_ref[...],
                           preferred_element_type=jnp.float32,
                           precision=lax.Precision.HIGHEST) * dinv


def _fin_body(acc_ref, deg_ref, b2_ref, out_ref):
    dinv = lax.rsqrt(deg_ref[...] + 1.0)
    out_ref[...] = acc_ref[...] * dinv + b2_ref[...]


_row_spec = pl.BlockSpec((BR, D), lambda i: (i, 0))
_col_spec = pl.BlockSpec((BR, 1), lambda i: (i, 0))
_w_spec = pl.BlockSpec((D, D), lambda i: (0, 0))
_b_spec = pl.BlockSpec((1, D), lambda i: (0, 0))
_GRID = (NPAD // BR,)
_row_out = jax.ShapeDtypeStruct((NPAD, D), jnp.float32)

_mm1_call = pl.pallas_call(
    _mm1_body, grid=_GRID,
    in_specs=[_row_spec, _w_spec, _col_spec],
    out_specs=_row_spec, out_shape=_row_out)

_mid_call = pl.pallas_call(
    _mid_body, grid=_GRID,
    in_specs=[_row_spec, _col_spec, _b_spec, _w_spec],
    out_specs=_row_spec, out_shape=_row_out)

_fin_call = pl.pallas_call(
    _fin_body, grid=_GRID,
    in_specs=[_row_spec, _col_spec, _b_spec],
    out_specs=_row_spec, out_shape=_row_out)


def _cat_halves(t):
    return jnp.concatenate(
        [t[0:NRH], t[SH_ROWS:SH_ROWS + NRH],
         jnp.zeros((NPAD - N, D), jnp.float32)])


def kernel(x, edge_index, W1, b1, W2, b2):
    src = edge_index[0].astype(jnp.int32)
    dst = edge_index[1].astype(jnp.int32)
    src_p = jnp.concatenate([src, jnp.zeros((EPAD - E,), jnp.int32)])
    dst_p = jnp.concatenate([dst, jnp.full((EPAD - E,), N, jnp.int32)])
    x_p = jnp.concatenate([x, jnp.zeros((NPAD - N, D), x.dtype)])
    zeros = jnp.zeros((DHR, D), jnp.float32)

    iota40 = jnp.arange(DHR, dtype=jnp.int32)
    deg2 = _deg_call(dst_p, zeros, iota40)
    deg_col = deg2.reshape(2, DHR * D)[:, :NRH].reshape(N)
    deg = jnp.concatenate([deg_col, jnp.zeros((NPAD - N,),
                                              jnp.float32)])[:, None]
    hs1 = _mm1_call(x_p, W1, deg)
    acc1 = _cat_halves(_edge_call(src_p, dst_p, hs1))
    hs2 = _mid_call(acc1, deg, b1.reshape(1, D), W2)
    acc2 = _cat_halves(_edge_call(src_p, dst_p, hs2))
    out = _fin_call(acc2, deg, b2.reshape(1, D))
    return out[:N]


# speedup vs baseline: 19.2652x; 1.0215x over previous
import dataclasses

import jax
import jax.numpy as jnp
from jax import lax
from jax.experimental import pallas as pl
from jax.experimental.pallas import tpu as pltpu
from jax.experimental.pallas import tpu_sc as plsc

N = 10000
D = 128
E = 320000
NPAD = 11000
EPAD = 327680
NRH = 5000
SH_TAB = 5120
STR = SH_TAB // 16
JUNK = SH_TAB - 1
JUNK_D = 5119
EPT = EPAD // 16
CHUNK = 128
NCHUNK = EPT // CHUNK
BR = 1000

_mesh = plsc.VectorSubcoreMesh(core_axis_name="c", subcore_axis_name="s")

_sc_params = pltpu.CompilerParams()
if "needs_layout_passes" in pltpu.CompilerParams.__dataclass_fields__:
    _sc_params = dataclasses.replace(_sc_params, needs_layout_passes=False)


DHR = 5120 // D


def _deg_body(dst_hbm, zeros_hbm, iota_hbm, deg_out, dst_all, hist, idxr,
              deg_sh):
    cid = lax.axis_index("c")
    sid = lax.axis_index("s")
    nbase = cid * NRH
    ebase = sid * EPT
    pltpu.sync_copy(dst_hbm.at[pl.ds(ebase, EPT)], dst_all)
    pltpu.sync_copy(zeros_hbm, hist)
    pltpu.sync_copy(iota_hbm, idxr)

    @pl.when(sid == 0)
    def _():
        pltpu.sync_copy(zeros_hbm, deg_sh)

    plsc.subcore_barrier()
    ones16 = jnp.ones((16,), jnp.float32)

    @pl.loop(0, EPT // 16)
    def _(g):
        d16 = dst_all[pl.ds(g * 16, 16)]
        local = d16 - nbase
        ok = (local >= 0) & (local < NRH)
        lcl = jnp.where(ok, local, JUNK_D)
        plsc.addupdate_scatter(hist, [lcl >> 7, lcl & 127], ones16)

    pltpu.sync_copy(hist, deg_sh.at[idxr], add=True)
    plsc.subcore_barrier()

    @pl.when(sid == 0)
    def _():
        pltpu.sync_copy(deg_sh, hist)
        pltpu.sync_copy(hist, deg_out.at[pl.ds(cid * DHR, DHR)])


@jax.jit
def _deg_call(dst, zeros, iota40):
    return pl.kernel(
        _deg_body,
        out_type=jax.ShapeDtypeStruct((2 * DHR, D), jnp.float32),
        mesh=_mesh,
        scratch_types=[
            pltpu.VMEM((EPT,), jnp.int32),
            pltpu.VMEM((DHR, D), jnp.float32),
            pltpu.VMEM((DHR,), jnp.int32),
            pltpu.VMEM_SHARED((DHR, D), jnp.float32),
        ],
        compiler_params=_sc_params,
    )(dst, zeros, iota40)


NBUF = 2
IB = 40
TRASH = EPT + CHUNK


def _edge_body(src_hbm, dst_hbm, hs_hbm, acc_out, src_all, dst_all, dstb,
               rows_v, buf_v, acc_sh, gsem, offr):
    cid = lax.axis_index("c")
    sid = lax.axis_index("s")
    nbase = cid * NRH
    ebase = sid * EPT
    rbase = sid * STR
    pltpu.sync_copy(src_hbm.at[pl.ds(ebase, EPT)], src_all.at[pl.ds(0, EPT)])
    pltpu.sync_copy(dst_hbm.at[pl.ds(ebase, EPT)], dst_all.at[pl.ds(0, EPT)])
    for b in range(STR // IB):
        pltpu.sync_copy(hs_hbm.at[pl.ds(nbase + rbase + b * IB, IB)], buf_v)
        pltpu.sync_copy(buf_v, acc_sh.at[pl.ds(rbase + b * IB, IB)])

    offr[0] = 0
    lane_g = lax.iota(jnp.int32, 16)

    @pl.loop(0, EPT // 16)
    def _(g):
        off = offr[0]
        s16 = src_all[pl.ds(g * 16, 16)]
        d16 = dst_all[pl.ds(g * 16, 16)]
        local = d16 - nbase
        ok = (local >= 0) & (local < NRH)
        oki = ok.astype(jnp.int32)
        pos = off + plsc.cumsum(oki) - oki
        tgt = jnp.where(ok, pos, TRASH)
        plsc.store_scatter(src_all, [tgt], s16)
        plsc.store_scatter(dst_all, [tgt], local)
        offr[0] = off + jnp.sum(oki)

    cnt = offr[0]
    lane = lax.iota(jnp.int32, 16)
    for k in range(CHUNK // 16):
        plsc.store_scatter(src_all, [cnt + k * 16 + lane],
                           jnp.zeros((16,), jnp.int32))
        plsc.store_scatter(dst_all, [cnt + k * 16 + lane],
                           jnp.full((16,), JUNK, jnp.int32))
    nch = (cnt + CHUNK - 1) // CHUNK
    plsc.subcore_barrier()

    def _gather(c, buf):
        pltpu.async_copy(hs_hbm.at[src_all.at[pl.ds(c * CHUNK, CHUNK)]],
                         rows_v.at[buf], gsem)

    for p in range(NBUF):
        @pl.when(p < nch)
        def _():
            _gather(p, p)

    @pl.loop(0, NCHUNK, step=NBUF)
    def _(c):
        for b in range(NBUF):
            @pl.when(c + b < nch)
            def _():
                for k in range(CHUNK // 16):
                    dstb[pl.ds(k * 16, 16)] = (
                        dst_all[pl.ds((c + b) * CHUNK + k * 16, 16)])
                pltpu.make_async_copy(hs_hbm.at[pl.ds(0, CHUNK)],
                                      rows_v.at[b], gsem).wait()
                pltpu.sync_copy(rows_v.at[b], acc_sh.at[dstb], add=True)

                @pl.when(c + b + NBUF < nch)
                def _():
                    _gather(c + b + NBUF, b)

    plsc.subcore_barrier()
    @pl.when(sid < 15)
    def _():
        for b in range(STR // IB):
            pltpu.sync_copy(acc_sh.at[pl.ds(rbase + b * IB, IB)], buf_v)
            pltpu.sync_copy(
                buf_v, acc_out.at[pl.ds(cid * NRH + rbase + b * IB, IB)])

    @pl.when(sid == 15)
    def _():
        for b in range((NRH - 15 * STR) // IB):
            pltpu.sync_copy(acc_sh.at[pl.ds(rbase + b * IB, IB)], buf_v)
            pltpu.sync_copy(
                buf_v, acc_out.at[pl.ds(cid * NRH + rbase + b * IB, IB)])


@jax.jit
def _edge_call(src, dst, hs):
    return pl.kernel(
        _edge_body,
        out_type=jax.ShapeDtypeStruct((NPAD, D), jnp.float32),
        mesh=_mesh,
        scratch_types=[
            pltpu.VMEM((EPT + CHUNK + 16,), jnp.int32),
            pltpu.VMEM((EPT + CHUNK + 16,), jnp.int32),
            pltpu.VMEM((CHUNK,), jnp.int32),
            pltpu.VMEM((NBUF, CHUNK, D), jnp.float32),
            pltpu.VMEM((IB, D), jnp.float32),
            pltpu.VMEM_SHARED((SH_TAB, D), jnp.float32),
            pltpu.SemaphoreType.DMA,
            pltpu.SMEM((1,), jnp.int32),
        ],
        compiler_params=_sc_params,
    )(src, dst, hs)


def _mm1_body(x_ref, w_ref, deg_ref, hs_ref):
    dinv = lax.rsqrt(deg_ref[...] + 1.0)
    hs_ref[...] = jnp.dot(x_ref[...], w_ref[...],
                          preferred_element_type=jnp.float32,
                          precision=lax.Precision.HIGHEST) * dinv


def _mid_body(acc_ref, deg_ref, b1_ref, w2_ref, hs2_ref):
    dinv = lax.rsqrt(deg_ref[...] + 1.0)
    h1 = jnp.maximum(acc_ref[...] * dinv + b1_ref[...], 0.0)
    hs2_ref[...] = jnp.dot(h1, w2_ref[...],
                           preferred_element_type=jnp.float32,
                           precision=lax.Precision.HIGHEST) * dinv


def _fin_body(acc_ref, deg_ref, b2_ref, out_ref):
    dinv = lax.rsqrt(deg_ref[...] + 1.0)
    out_ref[...] = acc_ref[...] * dinv + b2_ref[...]


_row_spec = pl.BlockSpec((BR, D), lambda i: (i, 0))
_col_spec = pl.BlockSpec((BR, 1), lambda i: (i, 0))
_w_spec = pl.BlockSpec((D, D), lambda i: (0, 0))
_b_spec = pl.BlockSpec((1, D), lambda i: (0, 0))
_acc_spec = _row_spec
_row_out = jax.ShapeDtypeStruct((NPAD, D), jnp.float32)

_mm1_call = pl.pallas_call(
    _mm1_body, grid=(NPAD // BR,),
    in_specs=[_row_spec, _w_spec, _col_spec],
    out_specs=_row_spec, out_shape=_row_out)

_mid_call = pl.pallas_call(
    _mid_body, grid=(NPAD // BR,),
    in_specs=[_acc_spec, _col_spec, _b_spec, _w_spec],
    out_specs=_row_spec, out_shape=_row_out)

_fin_call = pl.pallas_call(
    _fin_body, grid=(N // BR,),
    in_specs=[_acc_spec, _col_spec, _b_spec],
    out_specs=_row_spec, out_shape=jax.ShapeDtypeStruct((N, D), jnp.float32))


def kernel(x, edge_index, W1, b1, W2, b2):
    src = edge_index[0].astype(jnp.int32)
    dst = edge_index[1].astype(jnp.int32)
    src_p = jnp.concatenate([src, jnp.zeros((EPAD - E,), jnp.int32)])
    dst_p = jnp.concatenate([dst, jnp.full((EPAD - E,), N, jnp.int32)])
    x_p = jnp.concatenate([x, jnp.zeros((NPAD - N, D), x.dtype)])
    zeros = jnp.zeros((DHR, D), jnp.float32)

    iota40 = jnp.arange(DHR, dtype=jnp.int32)
    deg2 = _deg_call(dst_p, zeros, iota40)
    deg_col = deg2.reshape(2, DHR * D)[:, :NRH].reshape(N)
    deg = jnp.concatenate([deg_col, jnp.zeros((NPAD - N,),
                                              jnp.float32)])[:, None]
    hs1 = _mm1_call(x_p, W1, deg)
    acc1 = _edge_call(src_p, dst_p, hs1)
    hs2 = _mid_call(acc1, deg, b1.reshape(1, D), W2)
    acc2 = _edge_call(src_p, dst_p, hs2)
    return _fin_call(acc2, deg, b2.reshape(1, D))


# speedup vs baseline: 19.2927x; 1.0014x over previous
import dataclasses

import jax
import jax.numpy as jnp
from jax import lax
from jax.experimental import pallas as pl
from jax.experimental.pallas import tpu as pltpu
from jax.experimental.pallas import tpu_sc as plsc

N = 10000
D = 128
E = 320000
NPAD = 11000
EPAD = 327680
NRH = 5000
SH_TAB = 5120
STR = SH_TAB // 16
JUNK = SH_TAB - 1
JUNK_D = 5119
EPT = EPAD // 16
CHUNK = 128
NCHUNK = EPT // CHUNK
BR = 1000

_mesh = plsc.VectorSubcoreMesh(core_axis_name="c", subcore_axis_name="s")

_sc_params = pltpu.CompilerParams()
if "needs_layout_passes" in pltpu.CompilerParams.__dataclass_fields__:
    _sc_params = dataclasses.replace(_sc_params, needs_layout_passes=False)


DHR = 5120 // D


def _deg_body(dst_hbm, zeros_hbm, iota_hbm, deg_out, dst_all, hist, idxr,
              deg_sh):
    cid = lax.axis_index("c")
    sid = lax.axis_index("s")
    nbase = cid * NRH
    ebase = sid * EPT
    pltpu.sync_copy(dst_hbm.at[pl.ds(ebase, EPT)], dst_all)
    pltpu.sync_copy(zeros_hbm, hist)
    pltpu.sync_copy(iota_hbm, idxr)

    @pl.when(sid == 0)
    def _():
        pltpu.sync_copy(zeros_hbm, deg_sh)

    plsc.subcore_barrier()
    ones16 = jnp.ones((16,), jnp.float32)

    @pl.loop(0, EPT // 16)
    def _(g):
        d16 = dst_all[pl.ds(g * 16, 16)]
        local = d16 - nbase
        ok = (local >= 0) & (local < NRH)
        lcl = jnp.where(ok, local, JUNK_D)
        plsc.addupdate_scatter(hist, [lcl >> 7, lcl & 127], ones16)

    pltpu.sync_copy(hist, deg_sh.at[idxr], add=True)
    plsc.subcore_barrier()

    @pl.when(sid == 0)
    def _():
        pltpu.sync_copy(deg_sh, hist)
        pltpu.sync_copy(hist, deg_out.at[pl.ds(cid * DHR, DHR)])


@jax.jit
def _deg_call(dst, zeros, iota40):
    return pl.kernel(
        _deg_body,
        out_type=jax.ShapeDtypeStruct((2 * DHR, D), jnp.float32),
        mesh=_mesh,
        scratch_types=[
            pltpu.VMEM((EPT,), jnp.int32),
            pltpu.VMEM((DHR, D), jnp.float32),
            pltpu.VMEM((DHR,), jnp.int32),
            pltpu.VMEM_SHARED((DHR, D), jnp.float32),
        ],
        compiler_params=_sc_params,
    )(dst, zeros, iota40)


NBUF = 2
IB = 40
TRASH = EPT + CHUNK


def _edge_body(src_hbm, dst_hbm, hs_hbm, acc_out, src_all, dst_all, dstb,
               rows_v, buf_v, acc_sh, gsem, offr):
    cid = lax.axis_index("c")
    sid = lax.axis_index("s")
    nbase = cid * NRH
    ebase = sid * EPT
    rbase = sid * STR
    pltpu.sync_copy(src_hbm.at[pl.ds(ebase, EPT)], src_all.at[pl.ds(0, EPT)])
    pltpu.sync_copy(dst_hbm.at[pl.ds(ebase, EPT)], dst_all.at[pl.ds(0, EPT)])
    for b in range(STR // IB):
        pltpu.sync_copy(hs_hbm.at[pl.ds(nbase + rbase + b * IB, IB)], buf_v)
        pltpu.sync_copy(buf_v, acc_sh.at[pl.ds(rbase + b * IB, IB)])

    offr[0] = 0

    @pl.loop(0, EPT // 16)
    def _(g):
        off = offr[0]
        s16 = src_all[pl.ds(g * 16, 16)]
        d16 = dst_all[pl.ds(g * 16, 16)]
        local = d16 - nbase
        ok = (local >= 0) & (local < NRH)
        oki = ok.astype(jnp.int32)
        pos = off + plsc.cumsum(oki) - oki
        tgt = jnp.where(ok, pos, TRASH)
        plsc.store_scatter(src_all, [tgt], s16)
        plsc.store_scatter(dst_all, [tgt], local)
        offr[0] = off + jnp.sum(oki)

    cnt = offr[0]
    lane = lax.iota(jnp.int32, 16)
    for k in range(CHUNK // 16):
        plsc.store_scatter(src_all, [cnt + k * 16 + lane],
                           jnp.zeros((16,), jnp.int32))
        plsc.store_scatter(dst_all, [cnt + k * 16 + lane],
                           jnp.full((16,), JUNK, jnp.int32))
    nch = (cnt + CHUNK - 1) // CHUNK
    plsc.subcore_barrier()

    def _gather(c, buf):
        pltpu.async_copy(hs_hbm.at[src_all.at[pl.ds(c * CHUNK, CHUNK)]],
                         rows_v.at[buf], gsem)

    for p in range(NBUF):
        @pl.when(p < nch)
        def _():
            _gather(p, p)

    @pl.loop(0, NCHUNK, step=NBUF)
    def _(c):
        for b in range(NBUF):
            @pl.when(c + b < nch)
            def _():
                for k in range(CHUNK // 16):
                    dstb[pl.ds(k * 16, 16)] = (
                        dst_all[pl.ds((c + b) * CHUNK + k * 16, 16)])
                pltpu.make_async_copy(hs_hbm.at[pl.ds(0, CHUNK)],
                                      rows_v.at[b], gsem).wait()
                pltpu.sync_copy(rows_v.at[b], acc_sh.at[dstb], add=True)

                @pl.when(c + b + NBUF < nch)
                def _():
                    _gather(c + b + NBUF, b)

    plsc.subcore_barrier()
    @pl.when(sid < 15)
    def _():
        for b in range(STR // IB):
            pltpu.sync_copy(acc_sh.at[pl.ds(rbase + b * IB, IB)], buf_v)
            pltpu.sync_copy(
                buf_v, acc_out.at[pl.ds(cid * NRH + rbase + b * IB, IB)])

    @pl.when(sid == 15)
    def _():
        for b in range((NRH - 15 * STR) // IB):
            pltpu.sync_copy(acc_sh.at[pl.ds(rbase + b * IB, IB)], buf_v)
            pltpu.sync_copy(
                buf_v, acc_out.at[pl.ds(cid * NRH + rbase + b * IB, IB)])


@jax.jit
def _edge_call(src, dst, hs):
    return pl.kernel(
        _edge_body,
        out_type=jax.ShapeDtypeStruct((NPAD, D), jnp.float32),
        mesh=_mesh,
        scratch_types=[
            pltpu.VMEM((EPT + CHUNK + 16,), jnp.int32),
            pltpu.VMEM((EPT + CHUNK + 16,), jnp.int32),
            pltpu.VMEM((CHUNK,), jnp.int32),
            pltpu.VMEM((NBUF, CHUNK, D), jnp.float32),
            pltpu.VMEM((IB, D), jnp.float32),
            pltpu.VMEM_SHARED((SH_TAB, D), jnp.float32),
            pltpu.SemaphoreType.DMA,
            pltpu.SMEM((1,), jnp.int32),
        ],
        compiler_params=_sc_params,
    )(src, dst, hs)


def _mm1_body(x_ref, w_ref, deg_ref, hs_ref):
    dinv = lax.rsqrt(deg_ref[...] + 1.0)
    hs_ref[...] = jnp.dot(x_ref[...], w_ref[...],
                          preferred_element_type=jnp.float32,
                          precision=lax.Precision.HIGHEST) * dinv


def _mid_body(acc_ref, deg_ref, b1_ref, w2_ref, hs2_ref):
    dinv = lax.rsqrt(deg_ref[...] + 1.0)
    h1 = jnp.maximum(acc_ref[...] * dinv + b1_ref[...], 0.0)
    hs2_ref[...] = jnp.dot(h1, w2_ref[...],
                           preferred_element_type=jnp.float32,
                           precision=lax.Precision.HIGHEST) * dinv


def _fin_body(acc_ref, deg_ref, b2_ref, out_ref):
    dinv = lax.rsqrt(deg_ref[...] + 1.0)
    out_ref[...] = acc_ref[...] * dinv + b2_ref[...]


_row_spec = pl.BlockSpec((BR, D), lambda i: (i, 0))
_col_spec = pl.BlockSpec((BR, 1), lambda i: (i, 0))
_w_spec = pl.BlockSpec((D, D), lambda i: (0, 0))
_b_spec = pl.BlockSpec((1, D), lambda i: (0, 0))
_acc_spec = _row_spec
_row_out = jax.ShapeDtypeStruct((NPAD, D), jnp.float32)

_mm1_call = pl.pallas_call(
    _mm1_body, grid=(NPAD // BR,),
    in_specs=[_row_spec, _w_spec, _col_spec],
    out_specs=_row_spec, out_shape=_row_out)

_mid_call = pl.pallas_call(
    _mid_body, grid=(NPAD // BR,),
    in_specs=[_acc_spec, _col_spec, _b_spec, _w_spec],
    out_specs=_row_spec, out_shape=_row_out)

_fin_call = pl.pallas_call(
    _fin_body, grid=(N // BR,),
    in_specs=[_acc_spec, _col_spec, _b_spec],
    out_specs=_row_spec, out_shape=jax.ShapeDtypeStruct((N, D), jnp.float32))


def kernel(x, edge_index, W1, b1, W2, b2):
    src = edge_index[0].astype(jnp.int32)
    dst = edge_index[1].astype(jnp.int32)
    src_p = jnp.concatenate([src, jnp.zeros((EPAD - E,), jnp.int32)])
    dst_p = jnp.concatenate([dst, jnp.full((EPAD - E,), N, jnp.int32)])
    x_p = jnp.concatenate([x, jnp.zeros((NPAD - N, D), x.dtype)])
    zeros = jnp.zeros((DHR, D), jnp.float32)

    iota40 = jnp.arange(DHR, dtype=jnp.int32)
    deg2 = _deg_call(dst_p, zeros, iota40)
    deg_col = deg2.reshape(2, DHR * D)[:, :NRH].reshape(N)
    deg = jnp.concatenate([deg_col, jnp.zeros((NPAD - N,),
                                              jnp.float32)])[:, None]
    hs1 = _mm1_call(x_p, W1, deg)
    acc1 = _edge_call(src_p, dst_p, hs1)
    hs2 = _mid_call(acc1, deg, b1.reshape(1, D), W2)
    acc2 = _edge_call(src_p, dst_p, hs2)
    return _fin_call(acc2, deg, b2.reshape(1, D))
